# baseline (speedup 1.0000x reference)
import jax
import jax.numpy as jnp
from jax import lax
from jax.experimental import pallas as pl
from jax.experimental.pallas import tpu as pltpu


def kernel(indexes, table):
    num_indices = indexes.shape[0]
    dim = table.shape[1]
    half = num_indices // 2
    idx3 = indexes.astype(jnp.int32).reshape(2, 1, half)

    def body(idx_ref, table_ref, out_ref, rows_v, sem, osem):
        i = pl.program_id(0)

        def step(t, c):
            s = idx_ref[0, 0, t]
            pltpu.make_async_copy(
                table_ref.at[pl.ds(s, 1)], rows_v.at[pl.ds(t, 1)], sem
            ).start()
            return c

        lax.fori_loop(0, half, step, 0, unroll=8)
        pltpu.make_async_copy(
            table_ref.at[pl.ds(0, half)], rows_v, sem
        ).wait()
        ocp = pltpu.make_async_copy(
            rows_v, out_ref.at[pl.ds(i * half, half)], osem
        )
        ocp.start()
        ocp.wait()

    return pl.pallas_call(
        body,
        grid=(2,),
        in_specs=[
            pl.BlockSpec((1, 1, half), lambda i: (i, 0, 0),
                         memory_space=pltpu.SMEM),
            pl.BlockSpec(memory_space=pltpu.HBM),
        ],
        out_specs=pl.BlockSpec(memory_space=pltpu.HBM),
        out_shape=jax.ShapeDtypeStruct((num_indices, dim), jnp.float32),
        scratch_shapes=[
            pltpu.VMEM((half, dim), jnp.float32),
            pltpu.SemaphoreType.DMA,
            pltpu.SemaphoreType.DMA,
        ],
        compiler_params=pltpu.CompilerParams(
            dimension_semantics=("parallel",)
        ),
    )(idx3, table)


# speedup vs baseline: 1.0427x; 1.0427x over previous
import jax
import jax.numpy as jnp
from jax import lax
from jax.experimental import pallas as pl
from jax.experimental.pallas import tpu as pltpu
from jax.experimental.pallas import tpu_sc as plsc

_NC = 2
_NS = 16
_NW = _NC * _NS
_NSEM = 8
_SC_ROWS = 8192


def _sc_gather(idx, table, n_rows):
    dim = table.shape[1]
    b_per_w = n_rows // _NW
    idx2 = idx.reshape(_NW, b_per_w)
    mesh = plsc.VectorSubcoreMesh(core_axis_name="c", subcore_axis_name="s")

    @pl.kernel(
        out_type=jax.ShapeDtypeStruct((n_rows, dim), table.dtype),
        mesh=mesh,
        scratch_types=[
            pltpu.VMEM((b_per_w,), jnp.int32),
            pltpu.VMEM((b_per_w, dim), jnp.float32),
            pltpu.SemaphoreType.DMA,
        ]
        + [pltpu.SemaphoreType.DMA] * _NSEM,
    )
    def gather_kernel(table_hbm, idx_hbm, out_hbm, idx_v, rows_v, isem, *sems):
        wid = lax.axis_index("s") * _NC + lax.axis_index("c")
        pltpu.async_copy(idx_hbm.at[wid], idx_v, isem).wait()

        @pl.loop(0, b_per_w // 16)
        def _(j):
            base = j * 16
            v16 = idx_v[pl.ds(base, 16)]
            for k in range(16):
                pltpu.async_copy(
                    table_hbm.at[pl.ds(v16[k], 1)],
                    rows_v.at[pl.ds(base + k, 1)],
                    sems[k % _NSEM],
                )

        rows_per_sem = b_per_w // _NSEM
        for s in range(_NSEM):
            pltpu.make_async_copy(
                table_hbm.at[pl.ds(0, rows_per_sem)],
                rows_v.at[pl.ds(0, rows_per_sem)],
                sems[s],
            ).wait()
        pltpu.sync_copy(rows_v, out_hbm.at[pl.ds(wid * b_per_w, b_per_w)])

    return gather_kernel(table, idx2)


def _tc_gather(idx, table, n_rows):
    dim = table.shape[1]
    half = n_rows // 2
    idx3 = idx.reshape(2, 1, half)

    def body(idx_ref, table_ref, out_ref, rows_v, sem, osem):
        i = pl.program_id(0)

        def step(t, c):
            s = idx_ref[0, 0, t]
            pltpu.make_async_copy(
                table_ref.at[pl.ds(s, 1)], rows_v.at[pl.ds(t, 1)], sem
            ).start()
            return c

        lax.fori_loop(0, half, step, 0, unroll=8)
        pltpu.make_async_copy(table_ref.at[pl.ds(0, half)], rows_v, sem).wait()
        ocp = pltpu.make_async_copy(
            rows_v, out_ref.at[pl.ds(i * half, half)], osem
        )
        ocp.start()
        ocp.wait()

    return pl.pallas_call(
        body,
        grid=(2,),
        in_specs=[
            pl.BlockSpec((1, 1, half), lambda i: (i, 0, 0),
                         memory_space=pltpu.SMEM),
            pl.BlockSpec(memory_space=pltpu.HBM),
        ],
        out_specs=pl.BlockSpec(memory_space=pltpu.HBM),
        out_shape=jax.ShapeDtypeStruct((n_rows, dim), jnp.float32),
        scratch_shapes=[
            pltpu.VMEM((half, dim), jnp.float32),
            pltpu.SemaphoreType.DMA,
            pltpu.SemaphoreType.DMA,
        ],
        compiler_params=pltpu.CompilerParams(
            dimension_semantics=("parallel",)
        ),
    )(idx3, table)


def kernel(indexes, table):
    num_indices = indexes.shape[0]
    idx = indexes.astype(jnp.int32)

    @jax.jit
    def run(idx_arr, table_arr):
        sc_out = _sc_gather(idx_arr[:_SC_ROWS], table_arr, _SC_ROWS)
        tc_out = _tc_gather(
            idx_arr[_SC_ROWS:], table_arr, num_indices - _SC_ROWS
        )
        return jnp.concatenate([sc_out, tc_out], axis=0)

    return run(idx, table)


# speedup vs baseline: 1.1509x; 1.1038x over previous
import jax
import jax.numpy as jnp
from jax import lax
from jax.experimental import pallas as pl
from jax.experimental.pallas import tpu as pltpu
from jax.experimental.pallas import tpu_sc as plsc

_NC = 2
_NS = 16
_NW = _NC * _NS


def kernel(indexes, table):
    num_indices = indexes.shape[0]
    dim = table.shape[1]
    b_per_w = num_indices // _NW
    idx = indexes.astype(jnp.int32).reshape(_NW, b_per_w)

    mesh = plsc.VectorSubcoreMesh(core_axis_name="c", subcore_axis_name="s")

    @jax.jit
    def run(table_arr, idx_arr):
        @pl.kernel(
            out_type=jax.ShapeDtypeStruct((num_indices, dim), table_arr.dtype),
            mesh=mesh,
            scratch_types=[
                pltpu.VMEM((b_per_w,), jnp.int32),
                pltpu.VMEM((b_per_w, dim), jnp.float32),
                pltpu.SemaphoreType.DMA,
                pltpu.SemaphoreType.DMA,
            ],
        )
        def gather_kernel(
            table_hbm, idx_hbm, out_hbm, idx_v, rows_v, isem, sem
        ):
            wid = lax.axis_index("s") * _NC + lax.axis_index("c")
            pltpu.async_copy(idx_hbm.at[wid], idx_v, isem).wait()

            @pl.loop(0, b_per_w // 16)
            def _(j):
                base = j * 16
                v16 = idx_v[pl.ds(base, 16)]
                for k in range(16):
                    pltpu.async_copy(
                        table_hbm.at[pl.ds(v16[k], 1)],
                        rows_v.at[pl.ds(base + k, 1)],
                        sem,
                    )

            pltpu.make_async_copy(
                table_hbm.at[pl.ds(0, b_per_w)], rows_v, sem
            ).wait()
            pltpu.sync_copy(rows_v, out_hbm.at[pl.ds(wid * b_per_w, b_per_w)])

        return gather_kernel(table_arr, idx_arr)

    return run(table, idx)
